# baseline (speedup 1.0000x reference)
import functools

import jax
import jax.numpy as jnp
from jax import lax
from jax.experimental import pallas as pl
from jax.experimental.pallas import tpu as pltpu
from jax.experimental.pallas import tpu_sc as plsc

NUM_CORES = 2
NUM_SUBCORES = 16
NW = NUM_CORES * NUM_SUBCORES
LANES = 16


NH = 8


def _make_sc_hist(n_total, nbins, shift, xor_bias, prefix_shift):
    per_w = n_total // NW
    ch = min(per_w, 8192)
    n_chunks = per_w // ch
    n_vregs = ch // LANES
    mesh = plsc.VectorSubcoreMesh(
        core_axis_name="c", subcore_axis_name="s",
        num_cores=NUM_CORES, num_subcores=NUM_SUBCORES)

    unroll = 16

    def body(scores_hbm, prefix_hbm, out_hbm, buf, pvec, *hists):
        wid = lax.axis_index("s") * NUM_CORES + lax.axis_index("c")
        zeros = jnp.zeros((LANES,), jnp.int32)

        def zero(i, _):
            for h in hists:
                h[pl.ds(i * LANES, LANES)] = zeros
            return 0
        lax.fori_loop(0, nbins, zero, 0)

        pltpu.sync_copy(prefix_hbm, pvec)
        pv = pvec[...]
        lane = lax.iota(jnp.int32, LANES)
        ones = jnp.ones((LANES,), jnp.int32)
        base = wid * per_w
        su = min(unroll, n_vregs)

        for c in range(n_chunks):
            pltpu.sync_copy(scores_hbm.at[pl.ds(base + c * ch, ch)], buf)

            def step(k, _):
                for u in range(su):
                    vals = buf[pl.ds((k * su + u) * LANES, LANES)]
                    bits = plsc.bitcast(vals, jnp.int32)
                    key = bits ^ ((bits >> 31) & 0x7FFFFFFF)
                    b = (key >> shift) & (nbins - 1)
                    if xor_bias:
                        b = b ^ xor_bias
                    idx = b * LANES + lane
                    tgt = hists[u % NH]
                    if prefix_shift is None:
                        plsc.addupdate_scatter(tgt, [idx], ones)
                    else:
                        ok = (key >> prefix_shift) == pv
                        plsc.addupdate_scatter(tgt, [idx], ones, mask=ok)
                return 0
            lax.fori_loop(0, n_vregs // su, step, 0)

        for i, h in enumerate(hists):
            pltpu.sync_copy(h, out_hbm.at[wid * NH + i])

    return pl.kernel(
        body,
        out_type=jax.ShapeDtypeStruct((NW * NH, nbins * LANES), jnp.int32),
        mesh=mesh,
        compiler_params=pltpu.CompilerParams(needs_layout_passes=False),
        scratch_types=[
            pltpu.VMEM((ch,), jnp.float32),
            pltpu.VMEM((LANES,), jnp.int32),
        ] + [pltpu.VMEM((nbins * LANES,), jnp.int32) for _ in range(NH)],
    )


def _locate(hist, j):
    csum = jnp.cumsum(hist)
    b = jnp.argmax(csum > j).astype(jnp.int32)
    c_before = csum[b] - hist[b]
    return b, j - c_before


def _hsum(h, nbins):
    return h.reshape(NW * NH, nbins, LANES).sum(axis=(0, 2))


def _radix_select(flat, j):
    n = flat.shape[0]
    nb = 256
    zeros16 = jnp.zeros((LANES,), jnp.int32)

    h1 = _make_sc_hist(n, nb, 24, 128, None)(flat, zeros16)
    bin1, j2 = _locate(_hsum(h1, nb), j)
    pfx = bin1 - 128

    p2 = jnp.full((LANES,), pfx, jnp.int32)
    h2 = _make_sc_hist(n, nb, 16, 0, 24)(flat, p2)
    bin2, j3 = _locate(_hsum(h2, nb), j2)
    pfx = (pfx << 8) | bin2

    p3 = jnp.full((LANES,), pfx, jnp.int32)
    h3 = _make_sc_hist(n, nb, 8, 0, 16)(flat, p3)
    bin3, j4 = _locate(_hsum(h3, nb), j3)
    pfx = (pfx << 8) | bin3

    p4 = jnp.full((LANES,), pfx, jnp.int32)
    h4 = _make_sc_hist(n, nb, 0, 0, 8)(flat, p4)
    bin4, _ = _locate(_hsum(h4, nb), j4)

    vkey = (pfx << 8) | bin4
    vbits = vkey ^ ((vkey >> 31) & 0x7FFFFFFF)
    return lax.bitcast_convert_type(vbits, jnp.float32)


def _mask_body(wd_ref, ws_ref, bd_ref, bs_ref, thr_ref, wm_ref, bm_ref):
    scale = jnp.sqrt(jnp.float32(1.0) / jnp.float32(0.5))
    vw = thr_ref[0, 0]
    vb = thr_ref[0, 1]
    wm = jnp.where(ws_ref[...] >= vw, wd_ref[...] * scale, jnp.float32(0.0))
    wm_ref[...] = wm.astype(jnp.bfloat16)
    bm_ref[...] = jnp.where(bs_ref[...] >= vb, bd_ref[...] * scale,
                            jnp.float32(0.0))


def _matmul_body(x_ref, wm_ref, bm_ref, o_ref):
    xb = x_ref[...].astype(jnp.bfloat16)
    acc = lax.dot_general(
        xb, wm_ref[...], (((1,), (1,)), ((), ())),
        preferred_element_type=jnp.float32)
    o_ref[...] = acc + bm_ref[...].reshape(1, -1)


@jax.jit
def kernel(x, W_data, W_scores, b_data, b_scores):
    d = W_data.shape[0]
    n_w = d * d
    vw = _radix_select(W_scores.reshape(-1), n_w // 2)
    vb = _radix_select(b_scores, d // 2)

    thr = jnp.zeros((8, 128), jnp.float32).at[0, 0].set(vw).at[0, 1].set(vb)

    wm, bm = pl.pallas_call(
        _mask_body,
        out_shape=(
            jax.ShapeDtypeStruct((d, d), jnp.bfloat16),
            jax.ShapeDtypeStruct((16, d // 16), jnp.float32),
        ),
    )(W_data, W_scores, b_data.reshape(16, d // 16),
      b_scores.reshape(16, d // 16), thr)

    m = x.shape[0]
    blk = 512
    out = pl.pallas_call(
        _matmul_body,
        grid=(m // blk,),
        in_specs=[
            pl.BlockSpec((blk, d), lambda i: (i, 0)),
            pl.BlockSpec((d, d), lambda i: (0, 0)),
            pl.BlockSpec((1, d), lambda i: (0, 0)),
        ],
        out_specs=pl.BlockSpec((blk, d), lambda i: (i, 0)),
        out_shape=jax.ShapeDtypeStruct((m, d), jnp.float32),
    )(x, wm, bm.reshape(1, d))
    return out


# speedup vs baseline: 1.2700x; 1.2700x over previous
import functools

import jax
import jax.numpy as jnp
from jax import lax
from jax.experimental import pallas as pl
from jax.experimental.pallas import tpu as pltpu
from jax.experimental.pallas import tpu_sc as plsc

NUM_CORES = 2
NUM_SUBCORES = 16
NW = NUM_CORES * NUM_SUBCORES
LANES = 16


def _make_sc_hist(n_rows, n_cols, nbins, shift, xor_bias, prefix_shift):
    rows_w = n_rows // NW
    ch_rows = max(1, min(rows_w, 8192 // n_cols))
    n_chunks = rows_w // ch_rows
    n_vregs = n_cols // LANES
    mesh = plsc.VectorSubcoreMesh(
        core_axis_name="c", subcore_axis_name="s",
        num_cores=NUM_CORES, num_subcores=NUM_SUBCORES)

    unroll = 16

    def body(scores_hbm, prefix_hbm, out_hbm, buf, pvec, lhist):
        wid = lax.axis_index("s") * NUM_CORES + lax.axis_index("c")
        zeros = jnp.zeros((LANES,), jnp.int32)

        def zero(i, _):
            for u in range(min(unroll, nbins)):
                lhist[pl.ds((i * min(unroll, nbins) + u) * LANES, LANES)] = zeros
            return 0
        lax.fori_loop(0, nbins // min(unroll, nbins), zero, 0)

        pltpu.sync_copy(prefix_hbm, pvec)
        pv = pvec[...]
        lane = lax.iota(jnp.int32, LANES)
        ones = jnp.ones((LANES,), jnp.int32)
        base = wid * rows_w
        su = min(max(1, unroll // ch_rows), n_vregs)

        for c in range(n_chunks):
            pltpu.sync_copy(scores_hbm.at[pl.ds(base + c * ch_rows, ch_rows)],
                            buf)

            def step(k, _):
                for r in range(ch_rows):
                    for u in range(su):
                        vals = buf[r, pl.ds((k * su + u) * LANES, LANES)]
                        bits = plsc.bitcast(vals, jnp.int32)
                        key = bits ^ ((bits >> 31) & 0x7FFFFFFF)
                        b = (key >> shift) & (nbins - 1)
                        if xor_bias:
                            b = b ^ xor_bias
                        idx = b * LANES + lane
                        if prefix_shift is None:
                            plsc.addupdate_scatter(lhist, [idx], ones)
                        else:
                            ok = (key >> prefix_shift) == pv
                            plsc.addupdate_scatter(lhist, [idx], ones, mask=ok)
                return 0
            lax.fori_loop(0, n_vregs // su, step, 0)

        pltpu.sync_copy(lhist, out_hbm.at[wid])

    return pl.kernel(
        body,
        out_type=jax.ShapeDtypeStruct((NW, nbins * LANES), jnp.int32),
        mesh=mesh,
        compiler_params=pltpu.CompilerParams(needs_layout_passes=False),
        scratch_types=[
            pltpu.VMEM((ch_rows, n_cols), jnp.float32),
            pltpu.VMEM((LANES,), jnp.int32),
            pltpu.VMEM((nbins * LANES,), jnp.int32),
        ],
    )


def _locate(hist, j):
    csum = jnp.cumsum(hist)
    b = jnp.argmax(csum > j).astype(jnp.int32)
    c_before = csum[b] - hist[b]
    return b, j - c_before


def _hsum(h, nbins):
    return h.reshape(NW, nbins, LANES).sum(axis=(0, 2))


def _radix_select(arr2d, j):
    n_rows, n_cols = arr2d.shape
    zeros16 = jnp.zeros((LANES,), jnp.int32)

    h1 = _make_sc_hist(n_rows, n_cols, 4096, 20, 2048, None)(arr2d, zeros16)
    bin1, j2 = _locate(_hsum(h1, 4096), j)
    pfx = bin1 - 2048

    p2 = jnp.full((LANES,), pfx, jnp.int32)
    h2 = _make_sc_hist(n_rows, n_cols, 4096, 8, 0, 20)(arr2d, p2)
    bin2, j3 = _locate(_hsum(h2, 4096), j2)
    pfx = (pfx << 12) | bin2

    p3 = jnp.full((LANES,), pfx, jnp.int32)
    h3 = _make_sc_hist(n_rows, n_cols, 256, 0, 0, 8)(arr2d, p3)
    bin3, _ = _locate(_hsum(h3, 256), j3)

    vkey = (pfx << 8) | bin3
    vbits = vkey ^ ((vkey >> 31) & 0x7FFFFFFF)
    return lax.bitcast_convert_type(vbits, jnp.float32)


def _mask_body(wd_ref, ws_ref, bd_ref, bs_ref, thr_ref, wm_ref, bm_ref):
    scale = jnp.sqrt(jnp.float32(1.0) / jnp.float32(0.5))
    vw = thr_ref[0, 0]
    vb = thr_ref[0, 1]
    wm = jnp.where(ws_ref[...] >= vw, wd_ref[...] * scale, jnp.float32(0.0))
    wm_ref[...] = wm.astype(jnp.bfloat16)
    bm_ref[...] = jnp.where(bs_ref[...] >= vb, bd_ref[...] * scale,
                            jnp.float32(0.0))


def _matmul_body(x_ref, wm_ref, bm_ref, o_ref):
    xb = x_ref[...].astype(jnp.bfloat16)
    acc = lax.dot_general(
        xb, wm_ref[...], (((1,), (1,)), ((), ())),
        preferred_element_type=jnp.float32)
    o_ref[...] = acc + bm_ref[...].reshape(1, -1)


@jax.jit
def kernel(x, W_data, W_scores, b_data, b_scores):
    d = W_data.shape[0]
    n_w = d * d
    vw = _radix_select(W_scores, n_w // 2)
    vb = _radix_select(b_scores.reshape(NW, d // NW), d // 2)

    thr = jnp.zeros((8, 128), jnp.float32).at[0, 0].set(vw).at[0, 1].set(vb)

    wm, bm = pl.pallas_call(
        _mask_body,
        out_shape=(
            jax.ShapeDtypeStruct((d, d), jnp.bfloat16),
            jax.ShapeDtypeStruct((16, d // 16), jnp.float32),
        ),
    )(W_data, W_scores, b_data.reshape(16, d // 16),
      b_scores.reshape(16, d // 16), thr)

    m = x.shape[0]
    blk = 512
    out = pl.pallas_call(
        _matmul_body,
        grid=(m // blk,),
        in_specs=[
            pl.BlockSpec((blk, d), lambda i: (i, 0)),
            pl.BlockSpec((d, d), lambda i: (0, 0)),
            pl.BlockSpec((1, d), lambda i: (0, 0)),
        ],
        out_specs=pl.BlockSpec((blk, d), lambda i: (i, 0)),
        out_shape=jax.ShapeDtypeStruct((m, d), jnp.float32),
    )(x, wm, bm.reshape(1, d))
    return out


# speedup vs baseline: 1.2728x; 1.0022x over previous
import functools

import jax
import jax.numpy as jnp
from jax import lax
from jax.experimental import pallas as pl
from jax.experimental.pallas import tpu as pltpu
from jax.experimental.pallas import tpu_sc as plsc

NUM_CORES = 2
NUM_SUBCORES = 16
NW = NUM_CORES * NUM_SUBCORES
LANES = 16


def _make_sc_hist(n_rows, n_cols, nbins, shift, xor_bias, prefix_shift):
    rows_w = n_rows // NW
    ch_rows = max(1, min(rows_w, 8192 // n_cols))
    n_chunks = rows_w // ch_rows
    n_vregs = n_cols // LANES
    mesh = plsc.VectorSubcoreMesh(
        core_axis_name="c", subcore_axis_name="s",
        num_cores=NUM_CORES, num_subcores=NUM_SUBCORES)

    unroll = 16

    def body(scores_hbm, prefix_hbm, out_hbm, buf, pvec, lhist):
        wid = lax.axis_index("s") * NUM_CORES + lax.axis_index("c")
        zeros = jnp.zeros((LANES,), jnp.int32)

        def zero(i, _):
            for u in range(min(unroll, nbins)):
                lhist[pl.ds((i * min(unroll, nbins) + u) * LANES, LANES)] = zeros
            return 0
        lax.fori_loop(0, nbins // min(unroll, nbins), zero, 0)

        pltpu.sync_copy(prefix_hbm, pvec)
        pv = pvec[...]
        lane = lax.iota(jnp.int32, LANES)
        ones = jnp.ones((LANES,), jnp.int32)
        base = wid * rows_w
        su = min(max(1, unroll // ch_rows), n_vregs)

        for c in range(n_chunks):
            pltpu.sync_copy(scores_hbm.at[pl.ds(base + c * ch_rows, ch_rows)],
                            buf)

            def step(k, _):
                for r in range(ch_rows):
                    for u in range(su):
                        vals = buf[r, pl.ds((k * su + u) * LANES, LANES)]
                        bits = plsc.bitcast(vals, jnp.int32)
                        key = bits ^ ((bits >> 31) & 0x7FFFFFFF)
                        b = (key >> shift) & (nbins - 1)
                        if xor_bias:
                            b = b ^ xor_bias
                        idx = b * LANES + lane
                        if prefix_shift is None:
                            plsc.addupdate_scatter(lhist, [idx], ones)
                        else:
                            ok = (key >> prefix_shift) == pv
                            plsc.addupdate_scatter(lhist, [idx], ones, mask=ok)
                return 0
            lax.fori_loop(0, n_vregs // su, step, 0)

        pltpu.sync_copy(lhist, out_hbm.at[wid])

    return pl.kernel(
        body,
        out_type=jax.ShapeDtypeStruct((NW, nbins * LANES), jnp.int32),
        mesh=mesh,
        compiler_params=pltpu.CompilerParams(needs_layout_passes=False),
        scratch_types=[
            pltpu.VMEM((ch_rows, n_cols), jnp.float32),
            pltpu.VMEM((LANES,), jnp.int32),
            pltpu.VMEM((nbins * LANES,), jnp.int32),
        ],
    )


def _locate(hist, j):
    csum = jnp.cumsum(hist)
    b = jnp.argmax(csum > j).astype(jnp.int32)
    c_before = csum[b] - hist[b]
    return b, j - c_before


def _hsum(h, nbins):
    return h.reshape(NW, nbins, LANES).sum(axis=(0, 2))


def _radix_select(arr2d, j):
    n_rows, n_cols = arr2d.shape
    zeros16 = jnp.zeros((LANES,), jnp.int32)

    h1 = _make_sc_hist(n_rows, n_cols, 4096, 20, 2048, None)(arr2d, zeros16)
    bin1, j2 = _locate(_hsum(h1, 4096), j)
    pfx = bin1 - 2048

    p2 = jnp.full((LANES,), pfx, jnp.int32)
    h2 = _make_sc_hist(n_rows, n_cols, 4096, 8, 0, 20)(arr2d, p2)
    bin2, j3 = _locate(_hsum(h2, 4096), j2)
    pfx = (pfx << 12) | bin2

    p3 = jnp.full((LANES,), pfx, jnp.int32)
    h3 = _make_sc_hist(n_rows, n_cols, 256, 0, 0, 8)(arr2d, p3)
    bin3, _ = _locate(_hsum(h3, 256), j3)

    vkey = (pfx << 8) | bin3
    vbits = vkey ^ ((vkey >> 31) & 0x7FFFFFFF)
    return lax.bitcast_convert_type(vbits, jnp.float32)


def _mask_body(wd_ref, ws_ref, bd_ref, bs_ref, thr_ref, wm_ref, bm_ref):
    scale = jnp.sqrt(jnp.float32(1.0) / jnp.float32(0.5))
    vw = thr_ref[0, 0]
    vb = thr_ref[0, 1]
    wm = jnp.where(ws_ref[...] >= vw, wd_ref[...] * scale, jnp.float32(0.0))
    wm_ref[...] = wm.astype(jnp.bfloat16)
    bm_ref[...] = jnp.where(bs_ref[...] >= vb, bd_ref[...] * scale,
                            jnp.float32(0.0))


def _matmul_body(x_ref, wm_ref, bm_ref, o_ref):
    xb = x_ref[...].astype(jnp.bfloat16)
    acc = lax.dot_general(
        xb, wm_ref[...], (((1,), (1,)), ((), ())),
        preferred_element_type=jnp.float32)
    o_ref[...] = acc + bm_ref[...].reshape(1, -1)


@jax.jit
def kernel(x, W_data, W_scores, b_data, b_scores):
    d = W_data.shape[0]
    n_w = d * d
    vw = _radix_select(W_scores, n_w // 2)
    vb = _radix_select(b_scores.reshape(NW, d // NW), d // 2)

    thr = jnp.zeros((8, 128), jnp.float32).at[0, 0].set(vw).at[0, 1].set(vb)

    wm, bm = pl.pallas_call(
        _mask_body,
        out_shape=(
            jax.ShapeDtypeStruct((d, d), jnp.bfloat16),
            jax.ShapeDtypeStruct((16, d // 16), jnp.float32),
        ),
    )(W_data, W_scores, b_data.reshape(16, d // 16),
      b_scores.reshape(16, d // 16), thr)

    m = x.shape[0]
    blk = 1024
    out = pl.pallas_call(
        _matmul_body,
        grid=(m // blk,),
        in_specs=[
            pl.BlockSpec((blk, d), lambda i: (i, 0)),
            pl.BlockSpec((d, d), lambda i: (0, 0)),
            pl.BlockSpec((1, d), lambda i: (0, 0)),
        ],
        out_specs=pl.BlockSpec((blk, d), lambda i: (i, 0)),
        out_shape=jax.ShapeDtypeStruct((m, d), jnp.float32),
    )(x, wm, bm.reshape(1, d))
    return out
